# baseline (speedup 1.0000x reference)
import functools

import jax
import jax.numpy as jnp
from jax import lax
from jax.experimental import pallas as pl
from jax.experimental.pallas import tpu as pltpu
from jax.experimental.pallas import tpu_sc as plsc

_MAXC = 128
_SIGMA = 0.5
_LANES = 16
_NW = 32
_NC = 2


def _sc_gather_aabb(packed, faces, B, V3, Fn, FP):
    NSUB = _NW // B
    chunk = FP // NSUB
    G = chunk // _LANES
    V3p = -(-V3 // 8) * 8
    nrows = B * 15 + 3
    mesh = plsc.VectorSubcoreMesh(core_axis_name="c", subcore_axis_name="s")

    def body(packed_hbm, faces_hbm, out_hbm, vbuf, fbuf, comb, sem):
        b = lax.axis_index("c")
        p = lax.axis_index("s")
        fstart = jnp.minimum(p * chunk, Fn - chunk)
        pltpu.sync_copy(
            packed_hbm.at[pl.ds(pl.multiple_of(b * V3p, 8), V3)], vbuf)
        pltpu.sync_copy(faces_hbm.at[pl.ds(fstart, chunk)], fbuf)
        lane = lax.iota(jnp.int32, _LANES)
        row0 = b * 15

        def grp(g, carry):
            goff = pl.multiple_of(g * _LANES, _LANES)
            vals = {}
            for s in range(3):
                fid = plsc.load_gather(
                    fbuf, [lane + goff, jnp.full((_LANES,), s, jnp.int32)])
                comb[pl.ds((15 + s) * chunk + goff,
                           _LANES)] = plsc.bitcast(fid, jnp.float32)
                idx3 = fid * 3
                for c in range(3):
                    v = plsc.load_gather(vbuf, [idx3 + c])
                    vals[(s, c)] = v
                    comb[pl.ds((3 * s + c) * chunk + goff, _LANES)] = v
            for c in range(3):
                a0, a1, a2 = vals[(0, c)], vals[(1, c)], vals[(2, c)]
                comb[pl.ds((9 + c) * chunk + goff,
                           _LANES)] = jnp.minimum(jnp.minimum(a0, a1), a2)
                comb[pl.ds((12 + c) * chunk + goff,
                           _LANES)] = jnp.maximum(jnp.maximum(a0, a1), a2)
            return carry

        lax.fori_loop(0, G, grp, 0)
        handles = []
        for d in range(15):
            handles.append(pltpu.async_copy(
                comb.at[pl.ds(d * chunk, chunk)],
                out_hbm.at[pl.ds(pl.multiple_of((row0 + d) * FP + fstart, 8),
                                 chunk)],
                sem))
        for s in range(3):
            handles.append(pltpu.async_copy(
                comb.at[pl.ds((15 + s) * chunk, chunk)],
                out_hbm.at[pl.ds(pl.multiple_of((B * 15 + s) * FP + fstart, 8),
                                 chunk)],
                sem))
        for h in handles:
            h.wait()

    fn = pl.kernel(
        body,
        mesh=mesh,
        compiler_params=pltpu.CompilerParams(needs_layout_passes=False),
        out_type=jax.ShapeDtypeStruct((nrows * FP,), jnp.float32),
        scratch_types=[
            pltpu.VMEM((V3,), jnp.float32),
            pltpu.VMEM((chunk, 3), jnp.int32),
            pltpu.VMEM((18 * chunk,), jnp.float32),
            pltpu.SemaphoreType.DMA,
        ],
    )
    return fn(packed, faces)


def _tc_search_loss(comb_all, B, Fn):
    _, G2, L = comb_all.shape

    def body(comb_ref, out_ref):
        jidx = (lax.broadcasted_iota(jnp.int32, (G2, L), 0) * L
                + lax.broadcasted_iota(jnp.int32, (G2, L), 1))
        in_f = jidx < Fn
        triu = (lax.broadcasted_iota(jnp.int32, (L, L), 0)
                <= lax.broadcasted_iota(jnp.int32, (L, L), 1)
                ).astype(jnp.float32)
        tril_s = (lax.broadcasted_iota(jnp.int32, (G2, G2), 0)
                  > lax.broadcasted_iota(jnp.int32, (G2, G2), 1)
                  ).astype(jnp.float32)
        fa = [lax.bitcast_convert_type(comb_ref[B * 15 + t], jnp.int32)
              for t in range(3)]

        total = jnp.float32(0.0)
        for b in range(B):
            tri = [comb_ref[b * 15 + d] for d in range(9)]
            mn = [comb_ref[b * 15 + 9 + d] for d in range(3)]
            mx = [comb_ref[b * 15 + 12 + d] for d in range(3)]

            def cond(st):
                i, cnt, acc = st
                return jnp.logical_and(cnt < _MAXC, i < Fn)

            def step(st, tri=tri, mn=mn, mx=mx):
                i, cnt, acc = st
                sel = jidx == i

                def exf(x):
                    return jnp.sum(jnp.where(sel, x, 0.0))

                v = [exf(tri[d]) for d in range(9)]
                mni = [exf(mn[d]) for d in range(3)]
                mxi = [exf(mx[d]) for d in range(3)]
                fi = [jnp.sum(jnp.where(sel, fa[t], 0)) for t in range(3)]

                ok = jnp.logical_and(in_f, jidx > i)
                for d in range(3):
                    ok = ok & (mni[d] <= mx[d]) & (mxi[d] >= mn[d])
                sh = fa[0] == fi[0]
                for s in range(3):
                    for t in range(3):
                        if s == 0 and t == 0:
                            continue
                        sh = sh | (fa[t] == fi[s])
                valid = ok & jnp.logical_not(sh)
                validf = valid.astype(jnp.float32)
                crow = jnp.sum(valid.astype(jnp.int32))

                cx = (v[0] + v[3] + v[6]) / 3.0
                cy = (v[1] + v[4] + v[7]) / 3.0
                cz = (v[2] + v[5] + v[8]) / 3.0
                e0x, e0y, e0z = v[3] - v[0], v[4] - v[1], v[5] - v[2]
                e1x, e1y, e1z = v[6] - v[0], v[7] - v[1], v[8] - v[2]
                nx = e0y * e1z - e0z * e1y
                ny = e0z * e1x - e0x * e1z
                nz = e0x * e1y - e0y * e1x
                nn = jnp.sqrt(nx * nx + ny * ny + nz * nz) + 1e-9
                nhx, nhy, nhz = nx / nn, ny / nn, nz / nn
                r = jnp.float32(0.0)
                for k in range(3):
                    dxk = v[3 * k + 0] - cx
                    dyk = v[3 * k + 1] - cy
                    dzk = v[3 * k + 2] - cz
                    r = jnp.maximum(
                        r, jnp.sqrt(dxk * dxk + dyk * dyk + dzk * dzk))
                rinv = 1.0 / (r + 1e-9)

                lossj = jnp.zeros((G2, L), jnp.float32)
                for k in range(3):
                    dx = tri[3 * k + 0] - cx
                    dy = tri[3 * k + 1] - cy
                    dz = tri[3 * k + 2] - cz
                    h = dx * nhx + dy * nhy + dz * nhz
                    rx = dx - h * nhx
                    ry = dy - h * nhy
                    rz = dz - h * nhz
                    rho = jnp.sqrt(rx * rx + ry * ry + rz * rz)
                    phi = (jnp.maximum(1.0 - jnp.abs(h) / _SIGMA, 0.0)
                           * jnp.maximum(1.0 - rho * rinv, 0.0))
                    lossj = lossj + ((phi * dx) ** 2 + (phi * dy) ** 2
                                     + (phi * dz) ** 2)

                within = jnp.dot(validf, triu,
                                 preferred_element_type=jnp.float32)
                pref = jnp.dot(tril_s, within,
                               preferred_element_type=jnp.float32)
                cum = pref[:, L - 1:L] + within
                rem_f = (_MAXC - cnt).astype(jnp.float32)
                inc = valid & (cum <= rem_f)
                acc = acc + jnp.sum(jnp.where(inc, lossj, 0.0))
                cnt = jnp.minimum(cnt + crow, _MAXC)
                return i + 1, cnt, acc

            _, _, accb = lax.while_loop(
                cond, step,
                (jnp.int32(0), jnp.int32(0), jnp.float32(0.0)))
            total = total + accb
        out_ref[0, 0] = total

    out = pl.pallas_call(
        body,
        out_shape=jax.ShapeDtypeStruct((1, 1), jnp.float32),
        out_specs=pl.BlockSpec(memory_space=pltpu.SMEM),
    )(comb_all)
    return out[0, 0]


def kernel(vertices, faces):
    B, V, _ = vertices.shape
    Fn = faces.shape[0]
    FP = -(-Fn // (_NW * _LANES)) * (_NW * _LANES)
    V3 = V * 3

    V3p = -(-V3 // 8) * 8
    packed = jnp.pad(vertices.reshape(B, V3),
                     ((0, 0), (0, V3p - V3))).reshape(-1)

    comb = _sc_gather_aabb(packed, faces.astype(jnp.int32), B, V3, Fn, FP)
    comb_all = comb.reshape(B * 15 + 3, FP // 128, 128)
    return _tc_search_loss(comb_all, B, Fn)


# speedup vs baseline: 1.0470x; 1.0470x over previous
import functools

import jax
import jax.numpy as jnp
from jax import lax
from jax.experimental import pallas as pl
from jax.experimental.pallas import tpu as pltpu
from jax.experimental.pallas import tpu_sc as plsc

_MAXC = 128
_SIGMA = 0.5
_LANES = 16
_NW = 32
_NC = 2


def _sc_gather_aabb(packed, faces, B, V3, Fn, FP):
    NSUB = _NW // B
    chunk = FP // NSUB
    G = chunk // _LANES
    V3p = -(-V3 // 8) * 8
    nrows = B * 15 + 3
    mesh = plsc.VectorSubcoreMesh(core_axis_name="c", subcore_axis_name="s")

    def body(packed_hbm, faces_hbm, out_hbm, vbuf, fbuf, comb, sem):
        b = lax.axis_index("c")
        p = lax.axis_index("s")
        fstart = jnp.minimum(p * chunk, Fn - chunk)
        pltpu.sync_copy(
            packed_hbm.at[pl.ds(pl.multiple_of(b * V3p, 8), V3)], vbuf)
        pltpu.sync_copy(
            faces_hbm.at[pl.ds(pl.multiple_of(fstart * 3, 8), chunk * 3)],
            fbuf)
        lane = lax.iota(jnp.int32, _LANES)
        row0 = b * 15

        def grp(g, carry):
            goff = pl.multiple_of(g * _LANES, _LANES)
            vals = {}
            for s in range(3):
                fid = plsc.load_gather(fbuf, [(lane + goff) * 3 + s])
                comb[pl.ds((15 + s) * chunk + goff,
                           _LANES)] = plsc.bitcast(fid, jnp.float32)
                idx3 = fid * 3
                for c in range(3):
                    v = plsc.load_gather(vbuf, [idx3 + c])
                    vals[(s, c)] = v
                    comb[pl.ds((3 * s + c) * chunk + goff, _LANES)] = v
            for c in range(3):
                a0, a1, a2 = vals[(0, c)], vals[(1, c)], vals[(2, c)]
                comb[pl.ds((9 + c) * chunk + goff,
                           _LANES)] = jnp.minimum(jnp.minimum(a0, a1), a2)
                comb[pl.ds((12 + c) * chunk + goff,
                           _LANES)] = jnp.maximum(jnp.maximum(a0, a1), a2)
            return carry

        lax.fori_loop(0, G, grp, 0)
        handles = []
        for d in range(15):
            handles.append(pltpu.async_copy(
                comb.at[pl.ds(d * chunk, chunk)],
                out_hbm.at[pl.ds(pl.multiple_of((row0 + d) * FP + fstart, 8),
                                 chunk)],
                sem))
        for s in range(3):
            handles.append(pltpu.async_copy(
                comb.at[pl.ds((15 + s) * chunk, chunk)],
                out_hbm.at[pl.ds(pl.multiple_of((B * 15 + s) * FP + fstart, 8),
                                 chunk)],
                sem))
        for h in handles:
            h.wait()

    fn = pl.kernel(
        body,
        mesh=mesh,
        compiler_params=pltpu.CompilerParams(needs_layout_passes=False),
        out_type=jax.ShapeDtypeStruct((nrows * FP,), jnp.float32),
        scratch_types=[
            pltpu.VMEM((V3,), jnp.float32),
            pltpu.VMEM((3 * chunk,), jnp.int32),
            pltpu.VMEM((18 * chunk,), jnp.float32),
            pltpu.SemaphoreType.DMA,
        ],
    )
    return fn(packed, faces)


def _tc_search_loss(comb_all, B, Fn):
    _, G2, L = comb_all.shape

    def body(comb_ref, out_ref):
        jidx = (lax.broadcasted_iota(jnp.int32, (G2, L), 0) * L
                + lax.broadcasted_iota(jnp.int32, (G2, L), 1))
        in_f = jidx < Fn
        triu = (lax.broadcasted_iota(jnp.int32, (L, L), 0)
                <= lax.broadcasted_iota(jnp.int32, (L, L), 1)
                ).astype(jnp.float32)
        tril_s = (lax.broadcasted_iota(jnp.int32, (G2, G2), 0)
                  > lax.broadcasted_iota(jnp.int32, (G2, G2), 1)
                  ).astype(jnp.float32)
        fa = [lax.bitcast_convert_type(comb_ref[B * 15 + t], jnp.int32)
              for t in range(3)]

        total = jnp.float32(0.0)
        for b in range(B):
            tri = [comb_ref[b * 15 + d] for d in range(9)]
            mn = [comb_ref[b * 15 + 9 + d] for d in range(3)]
            mx = [comb_ref[b * 15 + 12 + d] for d in range(3)]

            def cond(st):
                i, cnt, acc = st
                return jnp.logical_and(cnt < _MAXC, i < Fn)

            def step(st, tri=tri, mn=mn, mx=mx):
                i, cnt, acc = st
                sel = jidx == i

                def exf(x):
                    return jnp.sum(jnp.where(sel, x, 0.0))

                v = [exf(tri[d]) for d in range(9)]
                mni = [exf(mn[d]) for d in range(3)]
                mxi = [exf(mx[d]) for d in range(3)]
                fi = [jnp.sum(jnp.where(sel, fa[t], 0)) for t in range(3)]

                ok = jnp.logical_and(in_f, jidx > i)
                for d in range(3):
                    ok = ok & (mni[d] <= mx[d]) & (mxi[d] >= mn[d])
                sh = fa[0] == fi[0]
                for s in range(3):
                    for t in range(3):
                        if s == 0 and t == 0:
                            continue
                        sh = sh | (fa[t] == fi[s])
                valid = ok & jnp.logical_not(sh)
                validf = valid.astype(jnp.float32)
                crow = jnp.sum(valid.astype(jnp.int32))

                cx = (v[0] + v[3] + v[6]) / 3.0
                cy = (v[1] + v[4] + v[7]) / 3.0
                cz = (v[2] + v[5] + v[8]) / 3.0
                e0x, e0y, e0z = v[3] - v[0], v[4] - v[1], v[5] - v[2]
                e1x, e1y, e1z = v[6] - v[0], v[7] - v[1], v[8] - v[2]
                nx = e0y * e1z - e0z * e1y
                ny = e0z * e1x - e0x * e1z
                nz = e0x * e1y - e0y * e1x
                nn = jnp.sqrt(nx * nx + ny * ny + nz * nz) + 1e-9
                nhx, nhy, nhz = nx / nn, ny / nn, nz / nn
                r = jnp.float32(0.0)
                for k in range(3):
                    dxk = v[3 * k + 0] - cx
                    dyk = v[3 * k + 1] - cy
                    dzk = v[3 * k + 2] - cz
                    r = jnp.maximum(
                        r, jnp.sqrt(dxk * dxk + dyk * dyk + dzk * dzk))
                rinv = 1.0 / (r + 1e-9)

                lossj = jnp.zeros((G2, L), jnp.float32)
                for k in range(3):
                    dx = tri[3 * k + 0] - cx
                    dy = tri[3 * k + 1] - cy
                    dz = tri[3 * k + 2] - cz
                    h = dx * nhx + dy * nhy + dz * nhz
                    rx = dx - h * nhx
                    ry = dy - h * nhy
                    rz = dz - h * nhz
                    rho = jnp.sqrt(rx * rx + ry * ry + rz * rz)
                    phi = (jnp.maximum(1.0 - jnp.abs(h) / _SIGMA, 0.0)
                           * jnp.maximum(1.0 - rho * rinv, 0.0))
                    lossj = lossj + ((phi * dx) ** 2 + (phi * dy) ** 2
                                     + (phi * dz) ** 2)

                within = jnp.dot(validf, triu,
                                 preferred_element_type=jnp.float32)
                pref = jnp.dot(tril_s, within,
                               preferred_element_type=jnp.float32)
                cum = pref[:, L - 1:L] + within
                rem_f = (_MAXC - cnt).astype(jnp.float32)
                inc = valid & (cum <= rem_f)
                acc = acc + jnp.sum(jnp.where(inc, lossj, 0.0))
                cnt = jnp.minimum(cnt + crow, _MAXC)
                return i + 1, cnt, acc

            _, _, accb = lax.while_loop(
                cond, step,
                (jnp.int32(0), jnp.int32(0), jnp.float32(0.0)))
            total = total + accb
        out_ref[0, 0] = total

    out = pl.pallas_call(
        body,
        out_shape=jax.ShapeDtypeStruct((1, 1), jnp.float32),
        out_specs=pl.BlockSpec(memory_space=pltpu.SMEM),
    )(comb_all)
    return out[0, 0]


def kernel(vertices, faces):
    B, V, _ = vertices.shape
    Fn = faces.shape[0]
    FP = -(-Fn // (_NW * _LANES)) * (_NW * _LANES)
    V3 = V * 3

    V3p = -(-V3 // 8) * 8
    packed = jnp.pad(vertices.reshape(B, V3),
                     ((0, 0), (0, V3p - V3))).reshape(-1)

    comb = _sc_gather_aabb(packed, faces.astype(jnp.int32).reshape(-1),
                           B, V3, Fn, FP)
    comb_all = comb.reshape(B * 15 + 3, FP // 128, 128)
    return _tc_search_loss(comb_all, B, Fn)


# speedup vs baseline: 1.0578x; 1.0103x over previous
import functools

import jax
import jax.numpy as jnp
from jax import lax
from jax.experimental import pallas as pl
from jax.experimental.pallas import tpu as pltpu
from jax.experimental.pallas import tpu_sc as plsc

_MAXC = 128
_SIGMA = 0.5
_LANES = 16
_NW = 32
_NC = 2


def _sc_gather_aabb(packed, faces, B, V3, Fn, FP):
    NSUB = _NW // B
    chunk = FP // NSUB
    G = chunk // _LANES
    V3p = -(-V3 // 8) * 8
    nrows = B * 9 + 3
    mesh = plsc.VectorSubcoreMesh(core_axis_name="c", subcore_axis_name="s")

    def body(packed_hbm, faces_hbm, out_hbm, vbuf, fbuf, comb, sem):
        b = lax.axis_index("c")
        p = lax.axis_index("s")
        fstart = jnp.minimum(p * chunk, Fn - chunk)
        pltpu.sync_copy(
            packed_hbm.at[pl.ds(pl.multiple_of(b * V3p, 8), V3)], vbuf)
        pltpu.sync_copy(
            faces_hbm.at[pl.ds(pl.multiple_of(fstart * 3, 8), chunk * 3)],
            fbuf)
        lane = lax.iota(jnp.int32, _LANES)
        row0 = b * 9

        def grp(g, carry):
            goff = pl.multiple_of(g * _LANES, _LANES)
            for s in range(3):
                fid = plsc.load_gather(fbuf, [(lane + goff) * 3 + s])
                comb[pl.ds((9 + s) * chunk + goff,
                           _LANES)] = plsc.bitcast(fid, jnp.float32)
                idx3 = fid * 3
                for c in range(3):
                    comb[pl.ds((3 * s + c) * chunk + goff,
                               _LANES)] = plsc.load_gather(vbuf, [idx3 + c])
            return carry

        lax.fori_loop(0, G, grp, 0)
        handles = []
        for d in range(9):
            handles.append(pltpu.async_copy(
                comb.at[pl.ds(d * chunk, chunk)],
                out_hbm.at[pl.ds(pl.multiple_of((row0 + d) * FP + fstart, 8),
                                 chunk)],
                sem))
        for s in range(3):
            handles.append(pltpu.async_copy(
                comb.at[pl.ds((9 + s) * chunk, chunk)],
                out_hbm.at[pl.ds(pl.multiple_of((B * 9 + s) * FP + fstart, 8),
                                 chunk)],
                sem))
        for h in handles:
            h.wait()

    fn = pl.kernel(
        body,
        mesh=mesh,
        compiler_params=pltpu.CompilerParams(needs_layout_passes=False),
        out_type=jax.ShapeDtypeStruct((nrows * FP,), jnp.float32),
        scratch_types=[
            pltpu.VMEM((V3,), jnp.float32),
            pltpu.VMEM((3 * chunk,), jnp.int32),
            pltpu.VMEM((12 * chunk,), jnp.float32),
            pltpu.SemaphoreType.DMA,
        ],
    )
    return fn(packed, faces)


def _tc_search_loss(comb_all, B, Fn):
    _, G2, L = comb_all.shape

    def body(comb_ref, out_ref):
        jidx = (lax.broadcasted_iota(jnp.int32, (G2, L), 0) * L
                + lax.broadcasted_iota(jnp.int32, (G2, L), 1))
        in_f = jidx < Fn
        triu = (lax.broadcasted_iota(jnp.int32, (L, L), 0)
                <= lax.broadcasted_iota(jnp.int32, (L, L), 1)
                ).astype(jnp.float32)
        tril_s = (lax.broadcasted_iota(jnp.int32, (G2, G2), 0)
                  > lax.broadcasted_iota(jnp.int32, (G2, G2), 1)
                  ).astype(jnp.float32)
        fa = [lax.bitcast_convert_type(comb_ref[B * 9 + t], jnp.int32)
              for t in range(3)]

        total = jnp.float32(0.0)
        for b in range(B):
            tri = [comb_ref[b * 9 + d] for d in range(9)]
            mn = [jnp.minimum(jnp.minimum(tri[d], tri[3 + d]), tri[6 + d])
                  for d in range(3)]
            mx = [jnp.maximum(jnp.maximum(tri[d], tri[3 + d]), tri[6 + d])
                  for d in range(3)]

            def cond(st):
                i, cnt, acc = st
                return jnp.logical_and(cnt < _MAXC, i < Fn)

            def step(st, tri=tri, mn=mn, mx=mx):
                i, cnt, acc = st
                sel = jidx == i

                def exf(x):
                    return jnp.sum(jnp.where(sel, x, 0.0))

                v = [exf(tri[d]) for d in range(9)]
                mni = [exf(mn[d]) for d in range(3)]
                mxi = [exf(mx[d]) for d in range(3)]
                fi = [jnp.sum(jnp.where(sel, fa[t], 0)) for t in range(3)]

                ok = jnp.logical_and(in_f, jidx > i)
                for d in range(3):
                    ok = ok & (mni[d] <= mx[d]) & (mxi[d] >= mn[d])
                sh = fa[0] == fi[0]
                for s in range(3):
                    for t in range(3):
                        if s == 0 and t == 0:
                            continue
                        sh = sh | (fa[t] == fi[s])
                valid = ok & jnp.logical_not(sh)
                validf = valid.astype(jnp.float32)
                crow = jnp.sum(valid.astype(jnp.int32))

                cx = (v[0] + v[3] + v[6]) / 3.0
                cy = (v[1] + v[4] + v[7]) / 3.0
                cz = (v[2] + v[5] + v[8]) / 3.0
                e0x, e0y, e0z = v[3] - v[0], v[4] - v[1], v[5] - v[2]
                e1x, e1y, e1z = v[6] - v[0], v[7] - v[1], v[8] - v[2]
                nx = e0y * e1z - e0z * e1y
                ny = e0z * e1x - e0x * e1z
                nz = e0x * e1y - e0y * e1x
                nn = jnp.sqrt(nx * nx + ny * ny + nz * nz) + 1e-9
                nhx, nhy, nhz = nx / nn, ny / nn, nz / nn
                r = jnp.float32(0.0)
                for k in range(3):
                    dxk = v[3 * k + 0] - cx
                    dyk = v[3 * k + 1] - cy
                    dzk = v[3 * k + 2] - cz
                    r = jnp.maximum(
                        r, jnp.sqrt(dxk * dxk + dyk * dyk + dzk * dzk))
                rinv = 1.0 / (r + 1e-9)

                lossj = jnp.zeros((G2, L), jnp.float32)
                for k in range(3):
                    dx = tri[3 * k + 0] - cx
                    dy = tri[3 * k + 1] - cy
                    dz = tri[3 * k + 2] - cz
                    h = dx * nhx + dy * nhy + dz * nhz
                    rx = dx - h * nhx
                    ry = dy - h * nhy
                    rz = dz - h * nhz
                    rho = jnp.sqrt(rx * rx + ry * ry + rz * rz)
                    phi = (jnp.maximum(1.0 - jnp.abs(h) / _SIGMA, 0.0)
                           * jnp.maximum(1.0 - rho * rinv, 0.0))
                    lossj = lossj + ((phi * dx) ** 2 + (phi * dy) ** 2
                                     + (phi * dz) ** 2)

                within = jnp.dot(validf, triu,
                                 preferred_element_type=jnp.float32)
                pref = jnp.dot(tril_s, within,
                               preferred_element_type=jnp.float32)
                cum = pref[:, L - 1:L] + within
                rem_f = (_MAXC - cnt).astype(jnp.float32)
                inc = valid & (cum <= rem_f)
                acc = acc + jnp.sum(jnp.where(inc, lossj, 0.0))
                cnt = jnp.minimum(cnt + crow, _MAXC)
                return i + 1, cnt, acc

            _, _, accb = lax.while_loop(
                cond, step,
                (jnp.int32(0), jnp.int32(0), jnp.float32(0.0)))
            total = total + accb
        out_ref[0, 0] = total

    out = pl.pallas_call(
        body,
        out_shape=jax.ShapeDtypeStruct((1, 1), jnp.float32),
        out_specs=pl.BlockSpec(memory_space=pltpu.SMEM),
    )(comb_all)
    return out[0, 0]


def kernel(vertices, faces):
    B, V, _ = vertices.shape
    Fn = faces.shape[0]
    FP = -(-Fn // (_NW * _LANES)) * (_NW * _LANES)
    V3 = V * 3

    V3p = -(-V3 // 8) * 8
    packed = jnp.pad(vertices.reshape(B, V3),
                     ((0, 0), (0, V3p - V3))).reshape(-1)

    comb = _sc_gather_aabb(packed, faces.astype(jnp.int32).reshape(-1),
                           B, V3, Fn, FP)
    comb_all = comb.reshape(B * 9 + 3, FP // 128, 128)
    return _tc_search_loss(comb_all, B, Fn)
